# Initial kernel scaffold; baseline (speedup 1.0000x reference)
import jax
import jax.numpy as jnp
from jax.experimental import pallas as pl


def kernel(x, edge_index, W1, a1s, a1d, b1, W2, a2s, a2d, b2, Wm, ams, amd, bm, Wv, avs, avd, bv, M1, c1, M2, c2, M3, c3):
    raise NotImplementedError("write your pallas kernel here")


# speedup vs baseline: 10.5201x; 10.5201x over previous
import functools

import jax
import jax.numpy as jnp
from jax import lax
from jax.experimental import pallas as pl
from jax.experimental.pallas import tpu as pltpu
from jax.experimental.pallas import tpu_sc as plsc

N = 10000
E = 320000
D_Z = 64

NC = 2
NS = 16
NW = NC * NS
EPW = E // NW
CHUNK = 80
NCHUNK = EPW // CHUNK
DCOL = 144
NPD = 10240
ROWS_PT = NPD // NS

_f32 = jnp.float32
_i32 = jnp.int32


def _sc_body(nsd, h_hbm, sd_hbm, src_hbm, dst_hbm, out_hbm,
             isrc_v, idst_v, sdg_s, sdg_d, rows_g, rows_s,
             out_sh, sem_s, sem_d, sem_r):
    c = lax.axis_index("c")
    s = lax.axis_index("s")
    wid = s * NC + c

    zero16 = jnp.zeros((16,), _f32)

    def _zero_row(r, carry):
        for c9 in range(DCOL // 16):
            rows_s[r, pl.ds(c9 * 16, 16)] = zero16
        return carry

    lax.fori_loop(0, CHUNK, _zero_row, 0)
    for k in range(ROWS_PT // CHUNK):
        pltpu.sync_copy(rows_s, out_sh.at[pl.ds(s * ROWS_PT + k * CHUNK, CHUNK)])

    def _issue(j, sl):
        pltpu.sync_copy(src_hbm.at[wid, j], isrc_v.at[sl])
        pltpu.sync_copy(dst_hbm.at[wid, j], idst_v.at[sl])
        pltpu.async_copy(sd_hbm.at[isrc_v.at[sl]], sdg_s.at[sl], sem_s.at[sl])
        pltpu.async_copy(sd_hbm.at[idst_v.at[sl]], sdg_d.at[sl], sem_d.at[sl])
        pltpu.async_copy(h_hbm.at[isrc_v.at[sl]], rows_g.at[sl], sem_r.at[sl])

    plsc.subcore_barrier()

    idx0 = jnp.zeros((16,), _i32)
    idx1 = jnp.full((16,), 1, _i32)
    lane = lax.iota(_i32, 16)
    mask0 = (lane == 0).astype(_f32)
    mask8 = (lane == 8).astype(_f32)

    _issue(0, 0)

    def _chunk(j, carry):
        p = lax.rem(j, 2)
        q = lax.rem(j + 1, 2)

        @pl.when(j + 1 < NCHUNK)
        def _():
            _issue(j + 1, q)

        pltpu.make_async_copy(sd_hbm.at[isrc_v.at[p]], sdg_s.at[p], sem_s.at[p]).wait()
        pltpu.make_async_copy(sd_hbm.at[idst_v.at[p]], sdg_d.at[p], sem_d.at[p]).wait()
        exs_m = []
        exs_v = []
        for t in range(CHUNK // 16):
            rowi = lane + (t * 16)
            sg = plsc.load_gather(sdg_s.at[p], [rowi, idx0])
            dg = plsc.load_gather(sdg_d.at[p], [rowi, idx1])
            v = sg + dg
            v = jnp.where(v >= 0.0, v, 0.2 * v)
            exs_m.append(jnp.exp(v))
            if nsd == 4:
                sg = plsc.load_gather(sdg_s.at[p], [rowi, jnp.full((16,), 2, _i32)])
                dg = plsc.load_gather(sdg_d.at[p], [rowi, jnp.full((16,), 3, _i32)])
                v = sg + dg
                v = jnp.where(v >= 0.0, v, 0.2 * v)
                exs_v.append(jnp.exp(v))
        pltpu.make_async_copy(h_hbm.at[isrc_v.at[p]], rows_g.at[p], sem_r.at[p]).wait()
        def _bcast(vec, l):
            return jnp.broadcast_to(
                jnp.sum(jnp.where(lane == l, vec, 0.0)), (16,))

        for t in range(CHUNK // 16):
            for l in range(16):
                e = t * 16 + l
                sm = _bcast(exs_m[t], l)
                if nsd == 4:
                    sv = _bcast(exs_v[t], l)
                    for c8 in range(4):
                        cs = pl.ds(c8 * 16, 16)
                        rows_s[e, cs] = rows_g[p, e, cs] * sm
                    for c8 in range(4, 8):
                        cs = pl.ds(c8 * 16, 16)
                        rows_s[e, cs] = rows_g[p, e, cs] * sv
                    rows_s[e, pl.ds(128, 16)] = sm * mask0 + sv * mask8
                else:
                    for c8 in range(8):
                        cs = pl.ds(c8 * 16, 16)
                        rows_s[e, cs] = rows_g[p, e, cs] * sm
                    rows_s[e, pl.ds(128, 16)] = sm * mask0
        pltpu.sync_copy(rows_s, out_sh.at[idst_v.at[p]], add=True)
        return carry

    lax.fori_loop(0, NCHUNK, _chunk, 0)

    plsc.subcore_barrier()
    pltpu.sync_copy(out_sh.at[pl.ds(s * ROWS_PT, ROWS_PT)],
                    out_hbm.at[c, pl.ds(s * ROWS_PT, ROWS_PT)])


def _make_sc_conv(nsd):
    mesh = plsc.VectorSubcoreMesh(core_axis_name="c", subcore_axis_name="s",
                                  num_cores=NC, num_subcores=NS)
    return pl.kernel(
        functools.partial(_sc_body, nsd),
        out_type=jax.ShapeDtypeStruct((NC, NPD, DCOL), _f32),
        mesh=mesh,
        scratch_types=[
            pltpu.VMEM((2, CHUNK), _i32),
            pltpu.VMEM((2, CHUNK), _i32),
            pltpu.VMEM((2, CHUNK, 16), _f32),
            pltpu.VMEM((2, CHUNK, 16), _f32),
            pltpu.VMEM((2, CHUNK, 128), _f32),
            pltpu.VMEM((CHUNK, DCOL), _f32),
            pltpu.VMEM_SHARED((NPD, DCOL), _f32),
            pltpu.SemaphoreType.DMA((2,)),
            pltpu.SemaphoreType.DMA((2,)),
            pltpu.SemaphoreType.DMA((2,)),
        ],
        compiler_params=pltpu.CompilerParams(use_tc_tiling_on_sc=False,
                                             needs_layout_passes=False),
    )


_B = 2000
_GRID = N // _B


def _pre_body(x_ref, w_ref, a_ref, h_ref, sd_ref):
    h = jnp.dot(x_ref[...], w_ref[...], preferred_element_type=_f32)
    h_ref[...] = h
    sd_ref[...] = jnp.dot(h, a_ref[...], preferred_element_type=_f32)


def _mid_body(p_ref, b_ref, w_ref, a_ref, h_ref, sd_ref):
    p = p_ref[0] + p_ref[1]
    den = p[:, 128:129] + 1e-16
    g = jnp.maximum(p[:, :128] / den + b_ref[...], 0.0)
    h = jnp.dot(g, w_ref[...], preferred_element_type=_f32)
    h_ref[...] = h
    sd_ref[...] = jnp.dot(h, a_ref[...], preferred_element_type=_f32)


def _post_body(p_ref, eps_ref, bm_ref, bv_ref, m1_ref, c1_ref, m2_ref,
               c2_ref, m3_ref, c3_ref, y_ref, mu_ref, lv_ref):
    p = p_ref[0] + p_ref[1]
    denm = p[:, 128:129] + 1e-16
    denv = p[:, 136:137] + 1e-16
    mu = p[:, :64] / denm + bm_ref[...]
    lv = p[:, 64:128] / denv + bv_ref[...]
    z = mu + jnp.exp(0.5 * lv) * eps_ref[...]
    y = jnp.maximum(jnp.dot(z, m1_ref[...], preferred_element_type=_f32)
                    + c1_ref[...], 0.0)
    y = jnp.maximum(jnp.dot(y, m2_ref[...], preferred_element_type=_f32)
                    + c2_ref[...], 0.0)
    y = jnp.dot(y, m3_ref[...], preferred_element_type=_f32) + c3_ref[...]
    y_ref[...] = y
    mu_ref[...] = mu
    lv_ref[...] = lv


def _full(shape):
    return pl.BlockSpec(shape, lambda i: (0,) * len(shape))


def _pre_call(x, w, a):
    nsd = a.shape[1]
    return pl.pallas_call(
        _pre_body,
        grid=(_GRID,),
        in_specs=[pl.BlockSpec((_B, 128), lambda i: (i, 0)),
                  _full((128, 128)), _full((128, nsd))],
        out_specs=[pl.BlockSpec((_B, 128), lambda i: (i, 0)),
                   pl.BlockSpec((_B, nsd), lambda i: (i, 0))],
        out_shape=[jax.ShapeDtypeStruct((N, 128), _f32),
                   jax.ShapeDtypeStruct((N, nsd), _f32)],
    )(x, w, a)


def _mid_call(p, b, w, a):
    nsd = a.shape[1]
    return pl.pallas_call(
        _mid_body,
        grid=(_GRID,),
        in_specs=[pl.BlockSpec((NC, _B, DCOL), lambda i: (0, i, 0)),
                  _full((1, 128)), _full((128, 128)), _full((128, nsd))],
        out_specs=[pl.BlockSpec((_B, 128), lambda i: (i, 0)),
                   pl.BlockSpec((_B, nsd), lambda i: (i, 0))],
        out_shape=[jax.ShapeDtypeStruct((N, 128), _f32),
                   jax.ShapeDtypeStruct((N, nsd), _f32)],
    )(p, b, w, a)


def _post_call(p, eps, bm, bv, m1, c1, m2, c2, m3, c3):
    return pl.pallas_call(
        _post_body,
        grid=(_GRID,),
        in_specs=[pl.BlockSpec((NC, _B, DCOL), lambda i: (0, i, 0)),
                  pl.BlockSpec((_B, 64), lambda i: (i, 0)),
                  _full((1, 64)), _full((1, 64)),
                  _full((64, 128)), _full((1, 128)),
                  _full((128, 64)), _full((1, 64)),
                  _full((64, 1)), _full((1, 1))],
        out_specs=[pl.BlockSpec((_B, 1), lambda i: (i, 0)),
                   pl.BlockSpec((_B, 64), lambda i: (i, 0)),
                   pl.BlockSpec((_B, 64), lambda i: (i, 0))],
        out_shape=[jax.ShapeDtypeStruct((N, 1), _f32),
                   jax.ShapeDtypeStruct((N, 64), _f32),
                   jax.ShapeDtypeStruct((N, 64), _f32)],
    )(p, eps, bm, bv, m1, c1, m2, c2, m3, c3)


def kernel(x, edge_index, W1, a1s, a1d, b1, W2, a2s, a2d, b2,
           Wm, ams, amd, bm, Wv, avs, avd, bv, M1, c1, M2, c2, M3, c3):
    src3 = edge_index[0].reshape(NW, NCHUNK, CHUNK)
    dst3 = edge_index[1].reshape(NW, NCHUNK, CHUNK)

    zp = jnp.zeros((128, 14), _f32)
    A1 = jnp.concatenate([jnp.stack([a1s, a1d], axis=1), zp], axis=1)
    A2 = jnp.concatenate([jnp.stack([a2s, a2d], axis=1), zp], axis=1)
    z64 = jnp.zeros((64,), _f32)
    A4 = jnp.concatenate([
        jnp.concatenate([
            jnp.stack([ams, amd, z64, z64], axis=1),
            jnp.stack([z64, z64, avs, avd], axis=1),
        ], axis=0),
        jnp.zeros((128, 12), _f32),
    ], axis=1)
    Wcat = jnp.concatenate([Wm, Wv], axis=1)
    eps = jax.random.normal(jax.random.key(42), (N, D_Z), _f32)

    sc2 = _make_sc_conv(2)
    sc4 = _make_sc_conv(4)

    h1, sd1 = _pre_call(x, W1, A1)
    p1 = sc2(h1, sd1, src3, dst3)
    h2, sd2 = _mid_call(p1, b1.reshape(1, 128), W2, A2)
    p2 = sc2(h2, sd2, src3, dst3)
    hc, sd4 = _mid_call(p2, b2.reshape(1, 128), Wcat, A4)
    p3 = sc4(hc, sd4, src3, dst3)
    y, mu, lv = _post_call(p3, eps, bm.reshape(1, 64), bv.reshape(1, 64),
                           M1, c1.reshape(1, 128), M2, c2.reshape(1, 64),
                           M3, c3.reshape(1, 1))
    return (y, mu, lv)
